# baseline (speedup 1.0000x reference)
import jax
import jax.numpy as jnp
from jax.experimental import pallas as pl
from jax.experimental.pallas import tpu as pltpu

N = 10000
NFEAT = 128
NHID = 128
NCLASS = 64

BI = 400
BH = BI // 2
GRID = N // BI


def _gcn_kernel(x_ref, adj_t_ref, adj_b_ref, w1_ref, b1_ref, w2_ref, b2_ref,
                out_ref, s1_ref, s2_ref):
    g = pl.program_id(0)

    @pl.when(g == 0)
    def _phase0():
        s1_ref[...] = jnp.dot(x_ref[...], w1_ref[...],
                              preferred_element_type=jnp.float32)

    @pl.when((g >= 1) & (g <= GRID))
    def _phase1():
        i = g - 1
        for half, ref in ((0, adj_t_ref), (1, adj_b_ref)):
            part = jnp.dot(ref[...], s1_ref[...],
                           preferred_element_type=jnp.float32)
            h = jnp.maximum(part + b1_ref[...], 0.0)
            s2_ref[pl.ds(i * BI + half * BH, BH), :] = jnp.dot(
                h, w2_ref[...], preferred_element_type=jnp.float32)

    @pl.when(g > GRID)
    def _phase2():
        for half, ref in ((0, adj_t_ref), (1, adj_b_ref)):
            o = jnp.dot(ref[...], s2_ref[...],
                        preferred_element_type=jnp.float32) + b2_ref[...]
            m = jnp.max(o, axis=1, keepdims=True)
            shifted = o - m
            lse = jnp.log(jnp.sum(jnp.exp(shifted), axis=1, keepdims=True))
            out_ref[pl.ds(half * BH, BH), :] = shifted - lse


def _adj_step(g):
    i1 = jnp.maximum(g - 1, 0)
    i2 = g - (GRID + 1)
    return jnp.where(g > GRID, i2, i1)


def _out_index(g):
    return (jnp.maximum(g - (GRID + 1), 0), 0)


@jax.jit
def kernel(x, adj, W1, b1, W2, b2):
    b1r = b1.reshape(1, NHID)
    b2r = b2.reshape(1, NCLASS)

    out = pl.pallas_call(
        _gcn_kernel,
        grid=(1 + 2 * GRID,),
        in_specs=[
            pl.BlockSpec((N, NFEAT), lambda g: (0, 0)),
            pl.BlockSpec((BH, N), lambda g: (2 * _adj_step(g), 0)),
            pl.BlockSpec((BH, N), lambda g: (2 * _adj_step(g) + 1, 0)),
            pl.BlockSpec((NFEAT, NHID), lambda g: (0, 0)),
            pl.BlockSpec((1, NHID), lambda g: (0, 0)),
            pl.BlockSpec((NHID, NCLASS), lambda g: (0, 0)),
            pl.BlockSpec((1, NCLASS), lambda g: (0, 0)),
        ],
        out_specs=pl.BlockSpec((BI, NCLASS), _out_index),
        out_shape=jax.ShapeDtypeStruct((N, NCLASS), jnp.float32),
        scratch_shapes=[
            pltpu.VMEM((N, NHID), jnp.float32),
            pltpu.VMEM((N, NCLASS), jnp.float32),
        ],
        compiler_params=pltpu.CompilerParams(
            dimension_semantics=("arbitrary",),
        ),
    )(x, adj, adj, W1, b1r, W2, b2r)

    return out


# speedup vs baseline: 1.0391x; 1.0391x over previous
import jax
import jax.numpy as jnp
from jax.experimental import pallas as pl
from jax.experimental.pallas import tpu as pltpu

N = 10000
NFEAT = 128
NHID = 128
NCLASS = 64

BI = 400
GRID = N // BI


def _gcn_kernel(x_ref, adj_ref, w1_ref, b1_ref, w2_ref, b2_ref,
                out_ref, s1_ref, s2_ref):
    g = pl.program_id(0)

    @pl.when(g == 0)
    def _phase0():
        s1_ref[...] = jnp.dot(x_ref[...], w1_ref[...],
                              preferred_element_type=jnp.float32)

    @pl.when((g >= 1) & (g <= GRID))
    def _phase1():
        i = g - 1
        part = jnp.dot(adj_ref[...], s1_ref[...],
                       preferred_element_type=jnp.float32)
        h = jnp.maximum(part + b1_ref[...], 0.0)
        s2_ref[pl.ds(i * BI, BI), :] = jnp.dot(
            h, w2_ref[...], preferred_element_type=jnp.float32)

    @pl.when(g > GRID)
    def _phase2():
        o = jnp.dot(adj_ref[...], s2_ref[...],
                    preferred_element_type=jnp.float32) + b2_ref[...]
        m = jnp.max(o, axis=1, keepdims=True)
        shifted = o - m
        lse = jnp.log(jnp.sum(jnp.exp(shifted), axis=1, keepdims=True))
        out_ref[...] = shifted - lse


def _adj_index(g):
    i1 = jnp.maximum(g - 1, 0)
    i2 = g - (GRID + 1)
    return (jnp.where(g > GRID, i2, i1), 0)


def _out_index(g):
    return (jnp.maximum(g - (GRID + 1), 0), 0)


@jax.jit
def kernel(x, adj, W1, b1, W2, b2):
    b1r = b1.reshape(1, NHID)
    b2r = b2.reshape(1, NCLASS)

    out = pl.pallas_call(
        _gcn_kernel,
        grid=(1 + 2 * GRID,),
        in_specs=[
            pl.BlockSpec((N, NFEAT), lambda g: (0, 0)),
            pl.BlockSpec((BI, N), _adj_index),
            pl.BlockSpec((NFEAT, NHID), lambda g: (0, 0)),
            pl.BlockSpec((1, NHID), lambda g: (0, 0)),
            pl.BlockSpec((NHID, NCLASS), lambda g: (0, 0)),
            pl.BlockSpec((1, NCLASS), lambda g: (0, 0)),
        ],
        out_specs=pl.BlockSpec((BI, NCLASS), _out_index),
        out_shape=jax.ShapeDtypeStruct((N, NCLASS), jnp.float32),
        scratch_shapes=[
            pltpu.VMEM((N, NHID), jnp.float32),
            pltpu.VMEM((N, NCLASS), jnp.float32),
        ],
        compiler_params=pltpu.CompilerParams(
            dimension_semantics=("arbitrary",),
        ),
    )(x, adj, W1, b1r, W2, b2r)

    return out


# speedup vs baseline: 1.0472x; 1.0079x over previous
import jax
import jax.numpy as jnp
from jax.experimental import pallas as pl
from jax.experimental.pallas import tpu as pltpu

N = 10000
NFEAT = 128
NHID = 128
NCLASS = 64

BI = 400
GRID = N // BI


def _gcn_kernel(x_ref, adj_ref, w1_ref, b1_ref, w2_ref, b2_ref,
                out_ref, s2_ref):
    g = pl.program_id(0)

    @pl.when(g < GRID)
    def _phase1():
        ax = jnp.dot(adj_ref[...], x_ref[...],
                     preferred_element_type=jnp.float32)
        h = jnp.maximum(
            jnp.dot(ax, w1_ref[...], preferred_element_type=jnp.float32)
            + b1_ref[...], 0.0)
        s2_ref[pl.ds(g * BI, BI), :] = jnp.dot(
            h, w2_ref[...], preferred_element_type=jnp.float32)

    @pl.when(g >= GRID)
    def _phase2():
        o = jnp.dot(adj_ref[...], s2_ref[...],
                    preferred_element_type=jnp.float32) + b2_ref[...]
        m = jnp.max(o, axis=1, keepdims=True)
        shifted = o - m
        lse = jnp.log(jnp.sum(jnp.exp(shifted), axis=1, keepdims=True))
        out_ref[...] = shifted - lse


def _adj_index(g):
    return (jnp.where(g >= GRID, g - GRID, g), 0)


def _out_index(g):
    return (jnp.maximum(g - GRID, 0), 0)


@jax.jit
def kernel(x, adj, W1, b1, W2, b2):
    b1r = b1.reshape(1, NHID)
    b2r = b2.reshape(1, NCLASS)

    out = pl.pallas_call(
        _gcn_kernel,
        grid=(2 * GRID,),
        in_specs=[
            pl.BlockSpec((N, NFEAT), lambda g: (0, 0)),
            pl.BlockSpec((BI, N), _adj_index),
            pl.BlockSpec((NFEAT, NHID), lambda g: (0, 0)),
            pl.BlockSpec((1, NHID), lambda g: (0, 0)),
            pl.BlockSpec((NHID, NCLASS), lambda g: (0, 0)),
            pl.BlockSpec((1, NCLASS), lambda g: (0, 0)),
        ],
        out_specs=pl.BlockSpec((BI, NCLASS), _out_index),
        out_shape=jax.ShapeDtypeStruct((N, NCLASS), jnp.float32),
        scratch_shapes=[
            pltpu.VMEM((N, NCLASS), jnp.float32),
        ],
        compiler_params=pltpu.CompilerParams(
            dimension_semantics=("arbitrary",),
        ),
    )(x, adj, W1, b1r, W2, b2r)

    return out


# speedup vs baseline: 1.0473x; 1.0000x over previous
import jax
import jax.numpy as jnp
from jax.experimental import pallas as pl
from jax.experimental.pallas import tpu as pltpu

N = 10000
NFEAT = 128
NHID = 128
NCLASS = 64

BI = 400
GRID = N // BI


def _gcn_kernel(x_ref, adj_ref, w1_ref, b1_ref, w2_ref, b2_ref,
                out_ref, s2_ref):
    g = pl.program_id(0)

    @pl.when(g < GRID)
    def _phase1():
        ax = jnp.dot(adj_ref[...], x_ref[...],
                     preferred_element_type=jnp.float32)
        h = jnp.maximum(
            jnp.dot(ax, w1_ref[...], preferred_element_type=jnp.float32)
            + b1_ref[...][None, :], 0.0)
        s2_ref[pl.ds(g * BI, BI), :] = jnp.dot(
            h, w2_ref[...], preferred_element_type=jnp.float32)

    @pl.when(g >= GRID)
    def _phase2():
        o = jnp.dot(adj_ref[...], s2_ref[...],
                    preferred_element_type=jnp.float32) + b2_ref[...][None, :]
        m = jnp.max(o, axis=1, keepdims=True)
        shifted = o - m
        lse = jnp.log(jnp.sum(jnp.exp(shifted), axis=1, keepdims=True))
        out_ref[...] = shifted - lse


def _adj_index(g):
    return (jnp.where(g >= GRID, g - GRID, g), 0)


def _out_index(g):
    return (jnp.maximum(g - GRID, 0), 0)


@jax.jit
def kernel(x, adj, W1, b1, W2, b2):
    out = pl.pallas_call(
        _gcn_kernel,
        grid=(2 * GRID,),
        in_specs=[
            pl.BlockSpec((N, NFEAT), lambda g: (0, 0)),
            pl.BlockSpec((BI, N), _adj_index),
            pl.BlockSpec((NFEAT, NHID), lambda g: (0, 0)),
            pl.BlockSpec((NHID,), lambda g: (0,)),
            pl.BlockSpec((NHID, NCLASS), lambda g: (0, 0)),
            pl.BlockSpec((NCLASS,), lambda g: (0,)),
        ],
        out_specs=pl.BlockSpec((BI, NCLASS), _out_index),
        out_shape=jax.ShapeDtypeStruct((N, NCLASS), jnp.float32),
        scratch_shapes=[
            pltpu.VMEM((N, NCLASS), jnp.float32),
        ],
        compiler_params=pltpu.CompilerParams(
            dimension_semantics=("arbitrary",),
        ),
    )(x, adj, W1, b1, W2, b2)

    return out
